# Initial kernel scaffold; baseline (speedup 1.0000x reference)
import functools

import jax
import jax.numpy as jnp
from jax import lax
from jax.experimental import pallas as pl
from jax.experimental.pallas import tpu as pltpu
from jax.experimental.pallas import tpu_sc as plsc

H = 512
W = 512
N = H * W
NC = 2
NS = 16
L = 16
NW = NC * NS
TPW = N // NW
SPW = N // NW
CH = 8192


def _round_half_even_nonneg(x):
    n = x.astype(jnp.int32)
    f = x - n.astype(jnp.float32)
    half = jnp.full((L,), 0.5, jnp.float32)
    up = (f > half) | ((f == half) & ((n & 1) == 1))
    return jnp.where(up, n + 1, n)


def _flat_body(fx_hbm, fy_hbm, flat_hbm, bx, by, bo, sem):
    wid = lax.axis_index("s") * NC + lax.axis_index("c")
    base = wid * SPW
    cx = pltpu.async_copy(fx_hbm.at[pl.ds(base, SPW)], bx, sem)
    cy = pltpu.async_copy(fy_hbm.at[pl.ds(base, SPW)], by, sem)
    cx.wait()
    cy.wait()

    @pl.loop(0, SPW // L)
    def _per_vreg(j):
        off = j * L
        lin = base + off + lax.iota(jnp.int32, L)
        xi = lin & (W - 1)
        yi = lax.shift_right_logical(lin, 9)
        px = xi.astype(jnp.float32) + bx[pl.ds(off, L)]
        py = yi.astype(jnp.float32) + by[pl.ds(off, L)]
        px = jnp.minimum(jnp.maximum(px, 0.0), float(W - 1))
        py = jnp.minimum(jnp.maximum(py, 0.0), float(H - 1))
        tx = _round_half_even_nonneg(px)
        ty = _round_half_even_nonneg(py)
        bo[pl.ds(off, L)] = lax.shift_left(ty, 9) | tx

    pltpu.sync_copy(bo, flat_hbm.at[pl.ds(base, SPW)])


def _any_i32(p):
    return jnp.max(jnp.where(p, jnp.full((L,), 1, jnp.int32),
                             jnp.zeros((L,), jnp.int32)))


def _warp_body(flat_hbm, d_hbm, i0_hbm, i1_hbm, i2_hbm, sr_hbm,
               o0_hbm, o1_hbm, o2_hbm,
               zbuf, acc0, acc1, acc2, cnt, bf, bd, b0, b1, b2, bsr, sem):
    wid = lax.axis_index("s") * NC + lax.axis_index("c")
    tbase = wid * TPW
    pltpu.sync_copy(sr_hbm, bsr)
    srv = bsr[...]

    big = jnp.full((L,), 1e30, jnp.float32)
    zero = jnp.zeros((L,), jnp.float32)

    @pl.loop(0, TPW // L)
    def _init(i):
        off = i * L
        zbuf[pl.ds(off, L)] = big
        acc0[pl.ds(off, L)] = zero
        acc1[pl.ds(off, L)] = zero
        acc2[pl.ds(off, L)] = zero
        cnt[pl.ds(off, L)] = zero

    @pl.loop(0, N // CH)
    def _p1(c):
        cbase = c * CH
        cf = pltpu.async_copy(flat_hbm.at[pl.ds(cbase, CH)], bf, sem)
        cd = pltpu.async_copy(d_hbm.at[pl.ds(cbase, CH)], bd, sem)
        cf.wait()
        cd.wait()

        @pl.loop(0, CH // L)
        def _vreg(j):
            off = j * L
            fl = bf[pl.ds(off, L)]
            dd = bd[pl.ds(off, L)]
            rel = fl - tbase
            m = (rel >= 0) & (rel < TPW)
            ridx = rel & (TPW - 1)

            def _body(_):
                cur = plsc.load_gather(zbuf, [ridx], mask=m)
                pend = m & (dd < cur)
                plsc.store_scatter(zbuf, [ridx], dd, mask=pend)
                cur2 = plsc.load_gather(zbuf, [ridx], mask=m)
                return _any_i32(m & (dd < cur2))

            lax.while_loop(lambda t: t > 0, _body, _any_i32(m))

    @pl.loop(0, N // CH)
    def _p2(c):
        cbase = c * CH
        cf = pltpu.async_copy(flat_hbm.at[pl.ds(cbase, CH)], bf, sem)
        cd = pltpu.async_copy(d_hbm.at[pl.ds(cbase, CH)], bd, sem)
        c0 = pltpu.async_copy(i0_hbm.at[pl.ds(cbase, CH)], b0, sem)
        c1 = pltpu.async_copy(i1_hbm.at[pl.ds(cbase, CH)], b1, sem)
        c2 = pltpu.async_copy(i2_hbm.at[pl.ds(cbase, CH)], b2, sem)
        cf.wait()
        cd.wait()
        c0.wait()
        c1.wait()
        c2.wait()

        @pl.loop(0, CH // L)
        def _vreg(j):
            off = j * L
            fl = bf[pl.ds(off, L)]
            dd = bd[pl.ds(off, L)]
            rel = fl - tbase
            m = (rel >= 0) & (rel < TPW)
            ridx = rel & (TPW - 1)

            @pl.when(_any_i32(m) > 0)
            def _():
                zm = plsc.load_gather(zbuf, [ridx], mask=m)
                ok = m & (dd <= zm + srv)
                one = jnp.where(ok, jnp.full((L,), 1.0, jnp.float32), zero)
                plsc.addupdate_scatter(cnt, [ridx], one, mask=ok)
                plsc.addupdate_scatter(acc0, [ridx], b0[pl.ds(off, L)], mask=ok)
                plsc.addupdate_scatter(acc1, [ridx], b1[pl.ds(off, L)], mask=ok)
                plsc.addupdate_scatter(acc2, [ridx], b2[pl.ds(off, L)], mask=ok)

    @pl.loop(0, TPW // L)
    def _fin(i):
        off = i * L
        inv = 1.0 / jnp.maximum(cnt[pl.ds(off, L)], 1.0)
        acc0[pl.ds(off, L)] = acc0[pl.ds(off, L)] * inv
        acc1[pl.ds(off, L)] = acc1[pl.ds(off, L)] * inv
        acc2[pl.ds(off, L)] = acc2[pl.ds(off, L)] * inv

    pltpu.sync_copy(acc0, o0_hbm.at[pl.ds(tbase, TPW)])
    pltpu.sync_copy(acc1, o1_hbm.at[pl.ds(tbase, TPW)])
    pltpu.sync_copy(acc2, o2_hbm.at[pl.ds(tbase, TPW)])


@jax.jit
def _run(fx, fy, d, i0, i1, i2, srv):
    mesh = plsc.VectorSubcoreMesh(core_axis_name="c", subcore_axis_name="s")

    flat = pl.kernel(
        _flat_body,
        out_type=jax.ShapeDtypeStruct((N,), jnp.int32),
        mesh=mesh,
        scratch_types=[
            pltpu.VMEM((SPW,), jnp.float32),
            pltpu.VMEM((SPW,), jnp.float32),
            pltpu.VMEM((SPW,), jnp.int32),
            pltpu.SemaphoreType.DMA,
        ],
    )(fx, fy)

    o0, o1, o2 = pl.kernel(
        _warp_body,
        out_type=(
            jax.ShapeDtypeStruct((N,), jnp.float32),
            jax.ShapeDtypeStruct((N,), jnp.float32),
            jax.ShapeDtypeStruct((N,), jnp.float32),
        ),
        mesh=mesh,
        scratch_types=[
            pltpu.VMEM((TPW,), jnp.float32),
            pltpu.VMEM((TPW,), jnp.float32),
            pltpu.VMEM((TPW,), jnp.float32),
            pltpu.VMEM((TPW,), jnp.float32),
            pltpu.VMEM((TPW,), jnp.float32),
            pltpu.VMEM((CH,), jnp.int32),
            pltpu.VMEM((CH,), jnp.float32),
            pltpu.VMEM((CH,), jnp.float32),
            pltpu.VMEM((CH,), jnp.float32),
            pltpu.VMEM((CH,), jnp.float32),
            pltpu.VMEM((L,), jnp.float32),
            pltpu.SemaphoreType.DMA,
        ],
    )(flat, d, i0, i1, i2, srv)

    out = jnp.stack([o0, o1, o2], axis=-1)
    return out.reshape(H, W, 3)


def kernel(img, flow, depth, same_range):
    fx = flow[0, :, :, 0].reshape(-1)
    fy = flow[0, :, :, 1].reshape(-1)
    d = depth.reshape(-1)
    i0 = img[:, :, 0].reshape(-1)
    i1 = img[:, :, 1].reshape(-1)
    i2 = img[:, :, 2].reshape(-1)
    srv = jnp.full((L,), same_range, jnp.float32)
    return _run(fx, fy, d, i0, i1, i2, srv)


# speedup vs baseline: 1.5486x; 1.5486x over previous
import functools

import jax
import jax.numpy as jnp
from jax import lax
from jax.experimental import pallas as pl
from jax.experimental.pallas import tpu as pltpu
from jax.experimental.pallas import tpu_sc as plsc

H = 512
W = 512
N = H * W
NC = 2
NS = 16
L = 16
NW = NC * NS
TPW = N // NW
SPW = N // NW
CH = 8192


def _round_half_even_nonneg(x):
    n = x.astype(jnp.int32)
    f = x - n.astype(jnp.float32)
    half = jnp.full((L,), 0.5, jnp.float32)
    up = (f > half) | ((f == half) & ((n & 1) == 1))
    return jnp.where(up, n + 1, n)


def _flat_body(fx_hbm, fy_hbm, flat_hbm, bx, by, bo, sem):
    wid = lax.axis_index("s") * NC + lax.axis_index("c")
    base = wid * SPW
    cx = pltpu.async_copy(fx_hbm.at[pl.ds(base, SPW)], bx, sem)
    cy = pltpu.async_copy(fy_hbm.at[pl.ds(base, SPW)], by, sem)
    cx.wait()
    cy.wait()

    @pl.loop(0, SPW // L)
    def _per_vreg(j):
        off = j * L
        lin = base + off + lax.iota(jnp.int32, L)
        xi = lin & (W - 1)
        yi = lax.shift_right_logical(lin, 9)
        px = xi.astype(jnp.float32) + bx[pl.ds(off, L)]
        py = yi.astype(jnp.float32) + by[pl.ds(off, L)]
        px = jnp.minimum(jnp.maximum(px, 0.0), float(W - 1))
        py = jnp.minimum(jnp.maximum(py, 0.0), float(H - 1))
        tx = _round_half_even_nonneg(px)
        ty = _round_half_even_nonneg(py)
        bo[pl.ds(off, L)] = lax.shift_left(ty, 9) | tx

    pltpu.sync_copy(bo, flat_hbm.at[pl.ds(base, SPW)])


def _any_f32(p):
    return jnp.sum(jnp.where(p, jnp.full((L,), 1.0, jnp.float32),
                             jnp.zeros((L,), jnp.float32)))


def _warp_body(flat_hbm, d_hbm, i0_hbm, i1_hbm, i2_hbm, sr_hbm,
               o0_hbm, o1_hbm, o2_hbm,
               zbuf, acc0, acc1, acc2, cnt, bf, bd, b0, b1, b2, bsr, sem):
    wid = lax.axis_index("s") * NC + lax.axis_index("c")
    tbase = wid * TPW
    pltpu.sync_copy(sr_hbm, bsr)
    srv = bsr[...]

    big = jnp.full((L,), 1e30, jnp.float32)
    zero = jnp.zeros((L,), jnp.float32)

    @pl.loop(0, TPW // L)
    def _init(i):
        off = i * L
        zbuf[pl.ds(off, L)] = big
        acc0[pl.ds(off, L)] = zero
        acc1[pl.ds(off, L)] = zero
        acc2[pl.ds(off, L)] = zero
        cnt[pl.ds(off, L)] = zero

    @pl.loop(0, N // CH)
    def _p1(c):
        cbase = c * CH
        cf = pltpu.async_copy(flat_hbm.at[pl.ds(cbase, CH)], bf, sem)
        cd = pltpu.async_copy(d_hbm.at[pl.ds(cbase, CH)], bd, sem)
        cf.wait()
        cd.wait()

        @pl.loop(0, CH // L)
        def _vreg(j):
            off = j * L
            fl = bf[pl.ds(off, L)]
            dd = bd[pl.ds(off, L)]
            rel = fl - tbase
            m = (rel >= 0) & (rel < TPW)
            ridx = rel & (TPW - 1)

            def _body(_):
                cur = plsc.load_gather(zbuf, [ridx], mask=m)
                pend = m & (dd < cur)
                plsc.store_scatter(zbuf, [ridx], dd, mask=pend)
                cur2 = plsc.load_gather(zbuf, [ridx], mask=m)
                return _any_f32(m & (dd < cur2))

            lax.while_loop(lambda t: t > 0.0, _body, _any_f32(m))

    @pl.loop(0, N // CH)
    def _p2(c):
        cbase = c * CH
        cf = pltpu.async_copy(flat_hbm.at[pl.ds(cbase, CH)], bf, sem)
        cd = pltpu.async_copy(d_hbm.at[pl.ds(cbase, CH)], bd, sem)
        c0 = pltpu.async_copy(i0_hbm.at[pl.ds(cbase, CH)], b0, sem)
        c1 = pltpu.async_copy(i1_hbm.at[pl.ds(cbase, CH)], b1, sem)
        c2 = pltpu.async_copy(i2_hbm.at[pl.ds(cbase, CH)], b2, sem)
        cf.wait()
        cd.wait()
        c0.wait()
        c1.wait()
        c2.wait()

        @pl.loop(0, CH // L)
        def _vreg(j):
            off = j * L
            fl = bf[pl.ds(off, L)]
            dd = bd[pl.ds(off, L)]
            rel = fl - tbase
            m = (rel >= 0) & (rel < TPW)
            ridx = rel & (TPW - 1)

            @pl.when(_any_f32(m) > 0.0)
            def _():
                zm = plsc.load_gather(zbuf, [ridx], mask=m)
                ok = m & (dd <= zm + srv)
                one = jnp.where(ok, jnp.full((L,), 1.0, jnp.float32), zero)
                plsc.addupdate_scatter(cnt, [ridx], one, mask=ok)
                plsc.addupdate_scatter(acc0, [ridx], b0[pl.ds(off, L)], mask=ok)
                plsc.addupdate_scatter(acc1, [ridx], b1[pl.ds(off, L)], mask=ok)
                plsc.addupdate_scatter(acc2, [ridx], b2[pl.ds(off, L)], mask=ok)

    @pl.loop(0, TPW // L)
    def _fin(i):
        off = i * L
        inv = 1.0 / jnp.maximum(cnt[pl.ds(off, L)], 1.0)
        acc0[pl.ds(off, L)] = acc0[pl.ds(off, L)] * inv
        acc1[pl.ds(off, L)] = acc1[pl.ds(off, L)] * inv
        acc2[pl.ds(off, L)] = acc2[pl.ds(off, L)] * inv

    pltpu.sync_copy(acc0, o0_hbm.at[pl.ds(tbase, TPW)])
    pltpu.sync_copy(acc1, o1_hbm.at[pl.ds(tbase, TPW)])
    pltpu.sync_copy(acc2, o2_hbm.at[pl.ds(tbase, TPW)])


@jax.jit
def _run(fx, fy, d, i0, i1, i2, srv):
    mesh = plsc.VectorSubcoreMesh(core_axis_name="c", subcore_axis_name="s")

    flat = pl.kernel(
        _flat_body,
        out_type=jax.ShapeDtypeStruct((N,), jnp.int32),
        mesh=mesh,
        compiler_params=pltpu.CompilerParams(needs_layout_passes=False),
        scratch_types=[
            pltpu.VMEM((SPW,), jnp.float32),
            pltpu.VMEM((SPW,), jnp.float32),
            pltpu.VMEM((SPW,), jnp.int32),
            pltpu.SemaphoreType.DMA,
        ],
    )(fx, fy)

    o0, o1, o2 = pl.kernel(
        _warp_body,
        out_type=(
            jax.ShapeDtypeStruct((N,), jnp.float32),
            jax.ShapeDtypeStruct((N,), jnp.float32),
            jax.ShapeDtypeStruct((N,), jnp.float32),
        ),
        mesh=mesh,
        compiler_params=pltpu.CompilerParams(needs_layout_passes=False),
        scratch_types=[
            pltpu.VMEM((TPW,), jnp.float32),
            pltpu.VMEM((TPW,), jnp.float32),
            pltpu.VMEM((TPW,), jnp.float32),
            pltpu.VMEM((TPW,), jnp.float32),
            pltpu.VMEM((TPW,), jnp.float32),
            pltpu.VMEM((CH,), jnp.int32),
            pltpu.VMEM((CH,), jnp.float32),
            pltpu.VMEM((CH,), jnp.float32),
            pltpu.VMEM((CH,), jnp.float32),
            pltpu.VMEM((CH,), jnp.float32),
            pltpu.VMEM((L,), jnp.float32),
            pltpu.SemaphoreType.DMA,
        ],
    )(flat, d, i0, i1, i2, srv)

    out = jnp.stack([o0, o1, o2], axis=-1)
    return out.reshape(H, W, 3)


def kernel(img, flow, depth, same_range):
    fx = flow[0, :, :, 0].reshape(-1)
    fy = flow[0, :, :, 1].reshape(-1)
    d = depth.reshape(-1)
    i0 = img[:, :, 0].reshape(-1)
    i1 = img[:, :, 1].reshape(-1)
    i2 = img[:, :, 2].reshape(-1)
    srv = jnp.full((L,), same_range, jnp.float32)
    return _run(fx, fy, d, i0, i1, i2, srv)
